# baseline (speedup 1.0000x reference)
import numpy as np
import jax
import jax.numpy as jnp
from jax.experimental import pallas as pl
from jax.experimental.pallas import tpu as pltpu

REG_MAX = 16
NM = 32
STRIDES = (8.0, 16.0, 32.0)
LEVELS = ((80, 80), (40, 40), (20, 20))
TOPK = 10
BOX_W, CLS_W, MASK_W, DFL_W = 7.5, 0.5, 2.5, 1.5
EPS = 1e-9
NA = sum(h * w for h, w in LEVELS)
NGT = 12
NPIX = 160 * 160
PIX_CHUNK = 6400
LEVEL_NA = tuple(h * w for h, w in LEVELS)
LEVEL_OFF = (0, LEVEL_NA[0], LEVEL_NA[0] + LEVEL_NA[1])


def _make_anchor_rows(level):
    (h, w), s = LEVELS[level], STRIDES[level]
    rows = np.zeros((8, h * w), dtype=np.float32)
    rows[0] = np.tile((np.arange(w, dtype=np.float32) + 0.5) * s, h)
    rows[1] = np.repeat((np.arange(h, dtype=np.float32) + 0.5) * s, w)
    rows[2] = np.full(h * w, s, dtype=np.float32)
    return jnp.asarray(rows)


def _loss_kernel(a3, a4, a5, gtb_ref, b3, b4, b5, c3, c4, c5, m3, m4, m5,
                 proto_ref, gtm_ref, out_ref, mcsel_s, wsel_s, agg_s,
                 cross_s, acc_s):
    cstep = pl.program_id(1)

    @pl.when(cstep == 0)
    def _assign_phase():
        acc_s[0] = 0.0
        _assign_body(a3, a4, a5, gtb_ref, b3, b4, b5, c3, c4, c5, m3, m4,
                     m5, out_ref, mcsel_s, wsel_s, agg_s, cross_s)

    mc_sel = mcsel_s[:, :]
    wsel = wsel_s[:, 0:1]
    pchunk = proto_ref[0]
    gchunk = gtm_ref[0]
    pm = jnp.dot(mc_sel, pchunk, preferred_element_type=jnp.float32)
    sp = jnp.maximum(pm, 0.0) + jnp.log1p(jnp.exp(-jnp.abs(pm)))
    acc_s[0] += jnp.sum(sp * wsel)
    cross_s[:, :] += jax.lax.dot_general(
        gchunk, pchunk, (((1,), (1,)), ((), ())),
        preferred_element_type=jnp.float32)

    @pl.when(cstep == NPIX // PIX_CHUNK - 1)
    def _finalize():
        l_msk = (acc_s[0] - jnp.sum(agg_s[:, :] * cross_s[:, :])
                 ) / float(NPIX)
        oiota = jax.lax.broadcasted_iota(jnp.int32, (1, 128), 1)
        out_ref[0, :, :] = (out_ref[0, :, :]
                            + jnp.where(oiota == 3, l_msk, 0.0))


def _assign_body(a3, a4, a5, gtb_ref, b3, b4, b5, c3, c4, c5, m3, m4, m5,
                 out_ref, mcsel_s, wsel_s, agg_s, cross_s):
    g_x1 = gtb_ref[0, :, 0:1]
    g_y1 = gtb_ref[0, :, 1:2]
    g_x2 = gtb_ref[0, :, 2:3]
    g_y2 = gtb_ref[0, :, 3:4]
    area_g = (g_x2 - g_x1) * (g_y2 - g_y1)

    proj = jax.lax.broadcasted_iota(jnp.int32, (REG_MAX, 1), 0).astype(
        jnp.float32)

    lv = []
    for aref, bref, cref in ((a3, b3, c3), (a4, b4, c4), (a5, b5, c5)):
        ax = aref[0:1, :]
        ay = aref[1:2, :]
        stv = aref[2:3, :]
        na = ax.shape[1]

        dists = []
        logps = []
        for k in range(4):
            lg = bref[0, REG_MAX * k:REG_MAX * (k + 1), :]
            mx = jnp.max(lg, axis=0, keepdims=True)
            ex = jnp.exp(lg - mx)
            s = jnp.sum(ex, axis=0, keepdims=True)
            dists.append(jnp.sum((ex / s) * proj, axis=0, keepdims=True))
            logps.append((lg - mx) - jnp.log(s))
        x1 = ax - dists[0] * stv
        y1 = ay - dists[1] * stv
        x2 = ax + dists[2] * stv
        y2 = ay + dists[3] * stv

        score = jax.nn.sigmoid(cref[0])

        iw = jnp.clip(jnp.minimum(g_x2, x2) - jnp.maximum(g_x1, x1), 0.0,
                      None)
        ih = jnp.clip(jnp.minimum(g_y2, y2) - jnp.maximum(g_y1, y1), 0.0,
                      None)
        inter = iw * ih
        area_d = (x2 - x1) * (y2 - y1)
        ious = inter / (area_g + area_d - inter + EPS)

        i2 = ious * ious
        align = score * (i2 * i2 * i2)
        in_gt = ((ax > g_x1) & (ax < g_x2) & (ay > g_y1) & (ay < g_y2))
        metric = jnp.where(in_gt, align, 0.0)
        lane_iota = jax.lax.broadcasted_iota(jnp.int32, (NGT, na), 1)
        lv.append(dict(ax=ax, ay=ay, stv=stv, logps=logps, x1=x1, y1=y1,
                       x2=x2, y2=y2, area_d=area_d, ious=ious,
                       metric=metric, iota=lane_iota, na=na))

    sels = [[], [], []]
    curs = [l['metric'] for l in lv]
    for _ in range(TOPK):
        mxs = [jnp.max(c, axis=1, keepdims=True) for c in curs]
        mx = jnp.maximum(jnp.maximum(mxs[0], mxs[1]), mxs[2])
        mps = [jnp.min(jnp.where(curs[i] == mx, lv[i]['iota'] + LEVEL_OFF[i],
                                 NA), axis=1, keepdims=True)
               for i in range(3)]
        mp = jnp.minimum(jnp.minimum(mps[0], mps[1]), mps[2])
        pos_ok = mx > 0.0
        for i in range(3):
            first = (lv[i]['iota'] + LEVEL_OFF[i]) == mp
            sels[i].append(jnp.where(first & pos_ok, 1.0, 0.0))
            curs[i] = jnp.where(first, -1.0, curs[i])

    l_box = 0.0
    l_cls = 0.0
    l_dfl = 0.0
    num_pos = 0.0
    wsel_parts = [None, None, None]
    for i in range(3):
        l = lv[i]
        na = l['na']
        mask_pos = sels[i][0]
        for k in range(1, TOPK):
            mask_pos = mask_pos + sels[i][k]
        fgf = jnp.minimum(jnp.sum(mask_pos, axis=0, keepdims=True), 1.0)
        num_pos = num_pos + jnp.sum(fgf)

        iou_m = jnp.where(mask_pos > 0.0, l['ious'], -1.0)
        best = iou_m[0:1, :]
        bidx = jnp.zeros((1, na), jnp.int32)
        for g in range(1, NGT):
            v = iou_m[g:g + 1, :]
            take = v > best
            best = jnp.where(take, v, best)
            bidx = jnp.where(take, g, bidx)
        sub_iota = jax.lax.broadcasted_iota(jnp.int32, (NGT, na), 0)
        oh = jnp.where(sub_iota == bidx, 1.0, 0.0)

        bg_x1 = jnp.sum(oh * g_x1, axis=0, keepdims=True)
        bg_y1 = jnp.sum(oh * g_y1, axis=0, keepdims=True)
        bg_x2 = jnp.sum(oh * g_x2, axis=0, keepdims=True)
        bg_y2 = jnp.sum(oh * g_y2, axis=0, keepdims=True)

        eiw = jnp.clip(jnp.minimum(bg_x2, l['x2'])
                       - jnp.maximum(bg_x1, l['x1']), 0.0, None)
        eih = jnp.clip(jnp.minimum(bg_y2, l['y2'])
                       - jnp.maximum(bg_y1, l['y1']), 0.0, None)
        einter = eiw * eih
        area_b = (bg_x2 - bg_x1) * (bg_y2 - bg_y1)
        iou_e = einter / (l['area_d'] + area_b - einter + EPS)
        l_box = l_box + jnp.sum((1.0 - iou_e) * fgf)

        clogit_ref = (c3, c4, c5)[i]
        x = clogit_ref[0]
        tgt = jnp.clip(iou_e, 0.0, 1.0)
        bce = (jnp.maximum(x, 0.0) - x * tgt
               + jnp.log1p(jnp.exp(-jnp.abs(x))))
        l_cls = l_cls + jnp.sum(bce * fgf)

        tvals = ((l['ax'] - bg_x1) / l['stv'], (l['ay'] - bg_y1) / l['stv'],
                 (bg_x2 - l['ax']) / l['stv'], (bg_y2 - l['ay']) / l['stv'])
        bin_iota = jax.lax.broadcasted_iota(jnp.int32, (REG_MAX, na), 0)
        for k in range(4):
            t = jnp.clip(tvals[k], 0.0, REG_MAX - 1e-6)
            tl = t.astype(jnp.int32)
            tr = jnp.minimum(tl + 1, REG_MAX - 1)
            at_b = tl >= REG_MAX - 1
            tr = jnp.where(at_b, tl, tr)
            wr = t - tl.astype(jnp.float32)
            wl = 1.0 - wr
            wr = jnp.where(at_b, 0.0, wr)
            wl = jnp.where(at_b, 1.0, wl)
            logp = l['logps'][k]
            ce_l = -jnp.sum(jnp.where(bin_iota == tl, logp, 0.0), axis=0,
                            keepdims=True)
            ce_r = -jnp.sum(jnp.where(bin_iota == tr, logp, 0.0), axis=0,
                            keepdims=True)
            l_dfl = l_dfl + jnp.sum((ce_l * wl + ce_r * wr) * fgf)

        wsel_parts[i] = jnp.concatenate(
            [jnp.sum(sels[i][k] * oh, axis=1, keepdims=True)
             for k in range(TOPK)], axis=0)

    wsel = wsel_parts[0] + wsel_parts[1] + wsel_parts[2]
    mc_sel = jnp.zeros((TOPK * NGT, NM), jnp.float32)
    for i, mref in enumerate((m3, m4, m5)):
        sel120 = jnp.concatenate(sels[i], axis=0)
        mc_sel = mc_sel + jax.lax.dot_general(
            sel120, mref[0], (((1,), (1,)), ((), ())),
            preferred_element_type=jnp.float32)

    eye12 = jnp.where(
        jax.lax.broadcasted_iota(jnp.int32, (NGT, NGT), 0)
        == jax.lax.broadcasted_iota(jnp.int32, (NGT, NGT), 1), 1.0, 0.0)
    tsel = jnp.concatenate([eye12] * TOPK, axis=0)
    agg = jax.lax.dot_general(tsel, mc_sel * wsel,
                              (((0,), (0,)), ((), ())),
                              preferred_element_type=jnp.float32)

    mcsel_s[:, :] = mc_sel
    wsel_s[:, :] = jnp.broadcast_to(wsel, (TOPK * NGT, 128))
    agg_s[:, :] = agg
    cross_s[:, :] = jnp.zeros((NGT, NM), jnp.float32)

    oiota = jax.lax.broadcasted_iota(jnp.int32, (1, 128), 1)
    vec = (jnp.where(oiota == 0, l_box, 0.0)
           + jnp.where(oiota == 1, l_cls, 0.0)
           + jnp.where(oiota == 2, l_dfl, 0.0)
           + jnp.where(oiota == 4, num_pos, 0.0))
    out_ref[0, :, :] = vec


def kernel(box_p3, box_p4, box_p5, cls_p3, cls_p4, cls_p5, mc_p3, mc_p4,
           mc_p5, proto, gt_boxes, gt_masks):
    B = box_p3.shape[0]
    boxes = [p.reshape(B, 4 * REG_MAX, -1) for p in (box_p3, box_p4, box_p5)]
    clss = [p.reshape(B, 1, -1) for p in (cls_p3, cls_p4, cls_p5)]
    mcs = [p.reshape(B, NM, -1) for p in (mc_p3, mc_p4, mc_p5)]
    proto_r = proto.reshape(B, NM, NPIX)
    gtm_r = gt_masks.reshape(B, NGT, NPIX)
    anchs = [_make_anchor_rows(i) for i in range(3)]

    in_specs = (
        [pl.BlockSpec((8, LEVEL_NA[i]), lambda b, c: (0, 0))
         for i in range(3)]
        + [pl.BlockSpec((1, NGT, 4), lambda b, c: (b, 0, 0))]
        + [pl.BlockSpec((1, 4 * REG_MAX, LEVEL_NA[i]), lambda b, c: (b, 0, 0))
           for i in range(3)]
        + [pl.BlockSpec((1, 1, LEVEL_NA[i]), lambda b, c: (b, 0, 0))
           for i in range(3)]
        + [pl.BlockSpec((1, NM, LEVEL_NA[i]), lambda b, c: (b, 0, 0))
           for i in range(3)]
        + [pl.BlockSpec((1, NM, PIX_CHUNK), lambda b, c: (b, 0, c)),
           pl.BlockSpec((1, NGT, PIX_CHUNK), lambda b, c: (b, 0, c))]
    )

    out = pl.pallas_call(
        _loss_kernel,
        grid=(B, NPIX // PIX_CHUNK),
        in_specs=in_specs,
        out_specs=pl.BlockSpec((1, 1, 128), lambda b, c: (b, 0, 0)),
        out_shape=jax.ShapeDtypeStruct((B, 1, 128), jnp.float32),
        scratch_shapes=[
            pltpu.VMEM((TOPK * NGT, NM), jnp.float32),
            pltpu.VMEM((TOPK * NGT, 128), jnp.float32),
            pltpu.VMEM((NGT, NM), jnp.float32),
            pltpu.VMEM((NGT, NM), jnp.float32),
            pltpu.SMEM((1,), jnp.float32),
        ],
        compiler_params=pltpu.CompilerParams(
            dimension_semantics=("arbitrary", "arbitrary")),
    )(anchs[0], anchs[1], anchs[2], gt_boxes, boxes[0], boxes[1], boxes[2],
      clss[0], clss[1], clss[2], mcs[0], mcs[1], mcs[2], proto_r, gtm_r)

    l_box = jnp.sum(out[:, 0, 0])
    l_cls = jnp.sum(out[:, 0, 1])
    l_dfl = jnp.sum(out[:, 0, 2])
    l_msk = jnp.sum(out[:, 0, 3])
    num_pos = jnp.sum(out[:, 0, 4])
    return (BOX_W * l_box / num_pos + CLS_W * l_cls / num_pos
            + MASK_W * l_msk / num_pos + DFL_W * l_dfl / (num_pos * 4.0))


# speedup vs baseline: 1.0147x; 1.0147x over previous
import numpy as np
import jax
import jax.numpy as jnp
from jax.experimental import pallas as pl
from jax.experimental.pallas import tpu as pltpu

REG_MAX = 16
NM = 32
STRIDES = (8.0, 16.0, 32.0)
LEVELS = ((80, 80), (40, 40), (20, 20))
TOPK = 10
BOX_W, CLS_W, MASK_W, DFL_W = 7.5, 0.5, 2.5, 1.5
EPS = 1e-9
NA = sum(h * w for h, w in LEVELS)
NGT = 12
NPIX = 160 * 160
PIX_CHUNK = 6400
LEVEL_NA = tuple(h * w for h, w in LEVELS)
LEVEL_OFF = (0, LEVEL_NA[0], LEVEL_NA[0] + LEVEL_NA[1])


def _make_anchor_rows(level):
    (h, w), s = LEVELS[level], STRIDES[level]
    rows = np.zeros((8, h * w), dtype=np.float32)
    rows[0] = np.tile((np.arange(w, dtype=np.float32) + 0.5) * s, h)
    rows[1] = np.repeat((np.arange(h, dtype=np.float32) + 0.5) * s, w)
    rows[2] = np.full(h * w, s, dtype=np.float32)
    return jnp.asarray(rows)


def _loss_kernel(a3, a4, a5, gtb_ref, b3, b4, b5, c3, c4, c5, m3, m4, m5,
                 proto_ref, gtm_ref, out_ref):
    g_x1 = gtb_ref[0, :, 0:1]
    g_y1 = gtb_ref[0, :, 1:2]
    g_x2 = gtb_ref[0, :, 2:3]
    g_y2 = gtb_ref[0, :, 3:4]
    area_g = (g_x2 - g_x1) * (g_y2 - g_y1)

    proj = jax.lax.broadcasted_iota(jnp.int32, (REG_MAX, 1), 0).astype(
        jnp.float32)

    lv = []
    for aref, bref, cref in ((a3, b3, c3), (a4, b4, c4), (a5, b5, c5)):
        ax = aref[0:1, :]
        ay = aref[1:2, :]
        stv = aref[2:3, :]
        na = ax.shape[1]

        dists = []
        logps = []
        for k in range(4):
            lg = bref[0, REG_MAX * k:REG_MAX * (k + 1), :]
            mx = jnp.max(lg, axis=0, keepdims=True)
            ex = jnp.exp(lg - mx)
            s = jnp.sum(ex, axis=0, keepdims=True)
            dists.append(jnp.sum((ex / s) * proj, axis=0, keepdims=True))
            logps.append((lg - mx) - jnp.log(s))
        x1 = ax - dists[0] * stv
        y1 = ay - dists[1] * stv
        x2 = ax + dists[2] * stv
        y2 = ay + dists[3] * stv

        score = jax.nn.sigmoid(cref[0])

        iw = jnp.clip(jnp.minimum(g_x2, x2) - jnp.maximum(g_x1, x1), 0.0,
                      None)
        ih = jnp.clip(jnp.minimum(g_y2, y2) - jnp.maximum(g_y1, y1), 0.0,
                      None)
        inter = iw * ih
        area_d = (x2 - x1) * (y2 - y1)
        ious = inter / (area_g + area_d - inter + EPS)

        i2 = ious * ious
        align = score * (i2 * i2 * i2)
        in_gt = ((ax > g_x1) & (ax < g_x2) & (ay > g_y1) & (ay < g_y2))
        metric = jnp.where(in_gt, align, 0.0)
        lane_iota = jax.lax.broadcasted_iota(jnp.int32, (NGT, na), 1)
        lv.append(dict(ax=ax, ay=ay, stv=stv, logps=logps, x1=x1, y1=y1,
                       x2=x2, y2=y2, area_d=area_d, ious=ious,
                       metric=metric, iota=lane_iota, na=na))

    sels = [[], [], []]
    curs = [l['metric'] for l in lv]
    for _ in range(TOPK):
        mxs = [jnp.max(c, axis=1, keepdims=True) for c in curs]
        mx = jnp.maximum(jnp.maximum(mxs[0], mxs[1]), mxs[2])
        mps = [jnp.min(jnp.where(curs[i] == mx, lv[i]['iota'] + LEVEL_OFF[i],
                                 NA), axis=1, keepdims=True)
               for i in range(3)]
        mp = jnp.minimum(jnp.minimum(mps[0], mps[1]), mps[2])
        pos_ok = mx > 0.0
        for i in range(3):
            first = (lv[i]['iota'] + LEVEL_OFF[i]) == mp
            sels[i].append(jnp.where(first & pos_ok, 1.0, 0.0))
            curs[i] = jnp.where(first, -1.0, curs[i])

    l_box = 0.0
    l_cls = 0.0
    l_dfl = 0.0
    num_pos = 0.0
    wsel_parts = [None, None, None]
    for i in range(3):
        l = lv[i]
        na = l['na']
        mask_pos = sels[i][0]
        for k in range(1, TOPK):
            mask_pos = mask_pos + sels[i][k]
        fgf = jnp.minimum(jnp.sum(mask_pos, axis=0, keepdims=True), 1.0)
        num_pos = num_pos + jnp.sum(fgf)

        iou_m = jnp.where(mask_pos > 0.0, l['ious'], -1.0)
        best = iou_m[0:1, :]
        bidx = jnp.zeros((1, na), jnp.int32)
        for g in range(1, NGT):
            v = iou_m[g:g + 1, :]
            take = v > best
            best = jnp.where(take, v, best)
            bidx = jnp.where(take, g, bidx)
        sub_iota = jax.lax.broadcasted_iota(jnp.int32, (NGT, na), 0)
        oh = jnp.where(sub_iota == bidx, 1.0, 0.0)

        bg_x1 = jnp.sum(oh * g_x1, axis=0, keepdims=True)
        bg_y1 = jnp.sum(oh * g_y1, axis=0, keepdims=True)
        bg_x2 = jnp.sum(oh * g_x2, axis=0, keepdims=True)
        bg_y2 = jnp.sum(oh * g_y2, axis=0, keepdims=True)

        eiw = jnp.clip(jnp.minimum(bg_x2, l['x2'])
                       - jnp.maximum(bg_x1, l['x1']), 0.0, None)
        eih = jnp.clip(jnp.minimum(bg_y2, l['y2'])
                       - jnp.maximum(bg_y1, l['y1']), 0.0, None)
        einter = eiw * eih
        area_b = (bg_x2 - bg_x1) * (bg_y2 - bg_y1)
        iou_e = einter / (l['area_d'] + area_b - einter + EPS)
        l_box = l_box + jnp.sum((1.0 - iou_e) * fgf)

        clogit_ref = (c3, c4, c5)[i]
        x = clogit_ref[0]
        tgt = jnp.clip(iou_e, 0.0, 1.0)
        bce = (jnp.maximum(x, 0.0) - x * tgt
               + jnp.log1p(jnp.exp(-jnp.abs(x))))
        l_cls = l_cls + jnp.sum(bce * fgf)

        tvals = ((l['ax'] - bg_x1) / l['stv'], (l['ay'] - bg_y1) / l['stv'],
                 (bg_x2 - l['ax']) / l['stv'], (bg_y2 - l['ay']) / l['stv'])
        bin_iota = jax.lax.broadcasted_iota(jnp.int32, (REG_MAX, na), 0)
        for k in range(4):
            t = jnp.clip(tvals[k], 0.0, REG_MAX - 1e-6)
            tl = t.astype(jnp.int32)
            tr = jnp.minimum(tl + 1, REG_MAX - 1)
            at_b = tl >= REG_MAX - 1
            tr = jnp.where(at_b, tl, tr)
            wr = t - tl.astype(jnp.float32)
            wl = 1.0 - wr
            wr = jnp.where(at_b, 0.0, wr)
            wl = jnp.where(at_b, 1.0, wl)
            logp = l['logps'][k]
            ce_l = -jnp.sum(jnp.where(bin_iota == tl, logp, 0.0), axis=0,
                            keepdims=True)
            ce_r = -jnp.sum(jnp.where(bin_iota == tr, logp, 0.0), axis=0,
                            keepdims=True)
            l_dfl = l_dfl + jnp.sum((ce_l * wl + ce_r * wr) * fgf)

        wsel_parts[i] = jnp.concatenate(
            [jnp.sum(sels[i][k] * oh, axis=1, keepdims=True)
             for k in range(TOPK)], axis=0)

    wsel = wsel_parts[0] + wsel_parts[1] + wsel_parts[2]
    mc_sel = jnp.zeros((TOPK * NGT, NM), jnp.float32)
    for i, mref in enumerate((m3, m4, m5)):
        sel120 = jnp.concatenate(sels[i], axis=0)
        mc_sel = mc_sel + jax.lax.dot_general(
            sel120, mref[0], (((1,), (1,)), ((), ())),
            preferred_element_type=jnp.float32)

    eye12 = jnp.where(
        jax.lax.broadcasted_iota(jnp.int32, (NGT, NGT), 0)
        == jax.lax.broadcasted_iota(jnp.int32, (NGT, NGT), 1), 1.0, 0.0)
    tsel = jnp.concatenate([eye12] * TOPK, axis=0)
    agg = jax.lax.dot_general(tsel, mc_sel * wsel,
                              (((0,), (0,)), ((), ())),
                              preferred_element_type=jnp.float32)
    acc = 0.0
    cross = jnp.zeros((NGT, NM), jnp.float32)
    for c in range(NPIX // PIX_CHUNK):
        pchunk = proto_ref[0, :, PIX_CHUNK * c:PIX_CHUNK * (c + 1)]
        gchunk = gtm_ref[0, :, PIX_CHUNK * c:PIX_CHUNK * (c + 1)]
        pm = jnp.dot(mc_sel, pchunk, preferred_element_type=jnp.float32)
        sp = jnp.maximum(pm, 0.0) + jnp.log1p(jnp.exp(-jnp.abs(pm)))
        acc = acc + jnp.sum(sp * wsel)
        cross = cross + jax.lax.dot_general(
            gchunk, pchunk, (((1,), (1,)), ((), ())),
            preferred_element_type=jnp.float32)
    l_msk = (acc - jnp.sum(agg * cross)) / float(NPIX)

    oiota = jax.lax.broadcasted_iota(jnp.int32, (1, 128), 1)
    vec = (jnp.where(oiota == 0, l_box, 0.0)
           + jnp.where(oiota == 1, l_cls, 0.0)
           + jnp.where(oiota == 2, l_dfl, 0.0)
           + jnp.where(oiota == 3, l_msk, 0.0)
           + jnp.where(oiota == 4, num_pos, 0.0))
    out_ref[0, :, :] = vec


def kernel(box_p3, box_p4, box_p5, cls_p3, cls_p4, cls_p5, mc_p3, mc_p4,
           mc_p5, proto, gt_boxes, gt_masks):
    B = box_p3.shape[0]
    boxes = [p.reshape(B, 4 * REG_MAX, -1) for p in (box_p3, box_p4, box_p5)]
    clss = [p.reshape(B, 1, -1) for p in (cls_p3, cls_p4, cls_p5)]
    mcs = [p.reshape(B, NM, -1) for p in (mc_p3, mc_p4, mc_p5)]
    proto_r = proto.reshape(B, NM, NPIX)
    gtm_r = gt_masks.reshape(B, NGT, NPIX)
    anchs = [_make_anchor_rows(i) for i in range(3)]

    in_specs = (
        [pl.BlockSpec((8, LEVEL_NA[i]), lambda b: (0, 0)) for i in range(3)]
        + [pl.BlockSpec((1, NGT, 4), lambda b: (b, 0, 0))]
        + [pl.BlockSpec((1, 4 * REG_MAX, LEVEL_NA[i]), lambda b: (b, 0, 0))
           for i in range(3)]
        + [pl.BlockSpec((1, 1, LEVEL_NA[i]), lambda b: (b, 0, 0))
           for i in range(3)]
        + [pl.BlockSpec((1, NM, LEVEL_NA[i]), lambda b: (b, 0, 0))
           for i in range(3)]
        + [pl.BlockSpec((1, NM, NPIX), lambda b: (b, 0, 0)),
           pl.BlockSpec((1, NGT, NPIX), lambda b: (b, 0, 0))]
    )

    out = pl.pallas_call(
        _loss_kernel,
        grid=(B,),
        in_specs=in_specs,
        out_specs=pl.BlockSpec((1, 1, 128), lambda b: (b, 0, 0)),
        out_shape=jax.ShapeDtypeStruct((B, 1, 128), jnp.float32),
        compiler_params=pltpu.CompilerParams(
            dimension_semantics=("parallel",),
            vmem_limit_bytes=100 * 1024 * 1024),
    )(anchs[0], anchs[1], anchs[2], gt_boxes, boxes[0], boxes[1], boxes[2],
      clss[0], clss[1], clss[2], mcs[0], mcs[1], mcs[2], proto_r, gtm_r)

    l_box = jnp.sum(out[:, 0, 0])
    l_cls = jnp.sum(out[:, 0, 1])
    l_dfl = jnp.sum(out[:, 0, 2])
    l_msk = jnp.sum(out[:, 0, 3])
    num_pos = jnp.sum(out[:, 0, 4])
    return (BOX_W * l_box / num_pos + CLS_W * l_cls / num_pos
            + MASK_W * l_msk / num_pos + DFL_W * l_dfl / (num_pos * 4.0))


# speedup vs baseline: 1.0299x; 1.0150x over previous
import numpy as np
import jax
import jax.numpy as jnp
from jax.experimental import pallas as pl
from jax.experimental.pallas import tpu as pltpu

REG_MAX = 16
NM = 32
STRIDES = (8.0, 16.0, 32.0)
LEVELS = ((80, 80), (40, 40), (20, 20))
TOPK = 10
BOX_W, CLS_W, MASK_W, DFL_W = 7.5, 0.5, 2.5, 1.5
EPS = 1e-9
NA = sum(h * w for h, w in LEVELS)
NGT = 12
NPIX = 160 * 160
PIX_CHUNK = 6400
LEVEL_NA = tuple(h * w for h, w in LEVELS)
LEVEL_OFF = (0, LEVEL_NA[0], LEVEL_NA[0] + LEVEL_NA[1])


def _make_anchor_rows(level):
    (h, w), s = LEVELS[level], STRIDES[level]
    rows = np.zeros((8, h * w), dtype=np.float32)
    rows[0] = np.tile((np.arange(w, dtype=np.float32) + 0.5) * s, h)
    rows[1] = np.repeat((np.arange(h, dtype=np.float32) + 0.5) * s, w)
    rows[2] = np.full(h * w, s, dtype=np.float32)
    return jnp.asarray(rows)


def _loss_kernel(a3, a4, a5, gtb_ref, b3, b4, b5, c3, c4, c5, m3, m4, m5,
                 proto_ref, gtm_ref, out_ref):
    g_x1 = gtb_ref[0, :, 0:1]
    g_y1 = gtb_ref[0, :, 1:2]
    g_x2 = gtb_ref[0, :, 2:3]
    g_y2 = gtb_ref[0, :, 3:4]
    area_g = (g_x2 - g_x1) * (g_y2 - g_y1)

    proj = jax.lax.broadcasted_iota(jnp.int32, (REG_MAX, 1), 0).astype(
        jnp.float32)

    lv = []
    for aref, bref, cref in ((a3, b3, c3), (a4, b4, c4), (a5, b5, c5)):
        ax = aref[0:1, :]
        ay = aref[1:2, :]
        stv = aref[2:3, :]
        na = ax.shape[1]

        dists = []
        lses = []
        for k in range(4):
            lg = bref[0, REG_MAX * k:REG_MAX * (k + 1), :]
            mx = jnp.max(lg, axis=0, keepdims=True)
            ex = jnp.exp(lg - mx)
            s = jnp.sum(ex, axis=0, keepdims=True)
            dists.append(jnp.sum((ex / s) * proj, axis=0, keepdims=True))
            lses.append(mx + jnp.log(s))
        x1 = ax - dists[0] * stv
        y1 = ay - dists[1] * stv
        x2 = ax + dists[2] * stv
        y2 = ay + dists[3] * stv

        score = jax.nn.sigmoid(cref[0])

        iw = jnp.clip(jnp.minimum(g_x2, x2) - jnp.maximum(g_x1, x1), 0.0,
                      None)
        ih = jnp.clip(jnp.minimum(g_y2, y2) - jnp.maximum(g_y1, y1), 0.0,
                      None)
        inter = iw * ih
        area_d = (x2 - x1) * (y2 - y1)
        ious = inter / (area_g + area_d - inter + EPS)

        i2 = ious * ious
        align = score * (i2 * i2 * i2)
        in_gt = ((ax > g_x1) & (ax < g_x2) & (ay > g_y1) & (ay < g_y2))
        metric = jnp.where(in_gt, align, 0.0)
        lane_iota = jax.lax.broadcasted_iota(jnp.int32, (NGT, na), 1)
        lv.append(dict(ax=ax, ay=ay, stv=stv, lses=lses, x1=x1, y1=y1,
                       x2=x2, y2=y2, area_d=area_d, ious=ious,
                       metric=metric, giota=lane_iota + LEVEL_OFF[len(lv)],
                       na=na))

    pvs = []
    pis = []
    pv = jnp.full((NGT, 1), jnp.inf, jnp.float32)
    pi = jnp.full((NGT, 1), -1, jnp.int32)
    for _ in range(TOPK):
        ems = []
        for i in range(3):
            m = lv[i]['metric']
            elig = (m < pv) | ((m == pv) & (lv[i]['giota'] > pi))
            ems.append(jnp.where(elig, m, -1.0))
        mx = jnp.maximum(
            jnp.maximum(jnp.max(ems[0], axis=1, keepdims=True),
                        jnp.max(ems[1], axis=1, keepdims=True)),
            jnp.max(ems[2], axis=1, keepdims=True))
        pos = jnp.minimum(
            jnp.minimum(
                jnp.min(jnp.where(ems[0] == mx, lv[0]['giota'], NA),
                        axis=1, keepdims=True),
                jnp.min(jnp.where(ems[1] == mx, lv[1]['giota'], NA),
                        axis=1, keepdims=True)),
            jnp.min(jnp.where(ems[2] == mx, lv[2]['giota'], NA),
                    axis=1, keepdims=True))
        pv, pi = mx, pos
        pvs.append(pv)
        pis.append(pi)
    pv10, pi10 = pvs[-1], pis[-1]

    l_box = 0.0
    l_cls = 0.0
    l_dfl = 0.0
    num_pos = 0.0
    for i in range(3):
        l = lv[i]
        na = l['na']
        m = l['metric']
        selected = (m > 0.0) & ((m > pv10)
                                | ((m == pv10) & (l['giota'] <= pi10)))
        fgf = jnp.max(jnp.where(selected, 1.0, 0.0), axis=0,
                      keepdims=True)
        num_pos = num_pos + jnp.sum(fgf)

        iou_m = jnp.where(selected, l['ious'], -1.0)
        best = iou_m[0:1, :]
        bidx = jnp.zeros((1, na), jnp.int32)
        for g in range(1, NGT):
            v = iou_m[g:g + 1, :]
            take = v > best
            best = jnp.where(take, v, best)
            bidx = jnp.where(take, g, bidx)
        sub_iota = jax.lax.broadcasted_iota(jnp.int32, (NGT, na), 0)
        oh = jnp.where(sub_iota == bidx, 1.0, 0.0)
        l['oh'] = oh

        bg_x1 = jnp.sum(oh * g_x1, axis=0, keepdims=True)
        bg_y1 = jnp.sum(oh * g_y1, axis=0, keepdims=True)
        bg_x2 = jnp.sum(oh * g_x2, axis=0, keepdims=True)
        bg_y2 = jnp.sum(oh * g_y2, axis=0, keepdims=True)

        eiw = jnp.clip(jnp.minimum(bg_x2, l['x2'])
                       - jnp.maximum(bg_x1, l['x1']), 0.0, None)
        eih = jnp.clip(jnp.minimum(bg_y2, l['y2'])
                       - jnp.maximum(bg_y1, l['y1']), 0.0, None)
        einter = eiw * eih
        area_b = (bg_x2 - bg_x1) * (bg_y2 - bg_y1)
        iou_e = einter / (l['area_d'] + area_b - einter + EPS)
        l_box = l_box + jnp.sum((1.0 - iou_e) * fgf)

        x = (c3, c4, c5)[i][0]
        tgt = jnp.clip(iou_e, 0.0, 1.0)
        bce = (jnp.maximum(x, 0.0) - x * tgt
               + jnp.log1p(jnp.exp(-jnp.abs(x))))
        l_cls = l_cls + jnp.sum(bce * fgf)

        bref = (b3, b4, b5)[i]
        tvals = ((l['ax'] - bg_x1) / l['stv'], (l['ay'] - bg_y1) / l['stv'],
                 (bg_x2 - l['ax']) / l['stv'], (bg_y2 - l['ay']) / l['stv'])
        bin_iota = jax.lax.broadcasted_iota(jnp.int32, (REG_MAX, na), 0)
        for k in range(4):
            t = jnp.clip(tvals[k], 0.0, REG_MAX - 1e-6)
            tl = t.astype(jnp.int32)
            tr = jnp.minimum(tl + 1, REG_MAX - 1)
            at_b = tl >= REG_MAX - 1
            tr = jnp.where(at_b, tl, tr)
            wr = t - tl.astype(jnp.float32)
            wl = 1.0 - wr
            wr = jnp.where(at_b, 0.0, wr)
            wl = jnp.where(at_b, 1.0, wl)
            lgk = bref[0, REG_MAX * k:REG_MAX * (k + 1), :]
            pick_l = jnp.sum(jnp.where(bin_iota == tl, lgk, 0.0), axis=0,
                             keepdims=True)
            pick_r = jnp.sum(jnp.where(bin_iota == tr, lgk, 0.0), axis=0,
                             keepdims=True)
            lse = l['lses'][k]
            l_dfl = l_dfl + jnp.sum(((lse - pick_l) * wl
                                     + (lse - pick_r) * wr) * fgf)

    mcw_rows = []
    nv = 0.0
    for k in range(TOPK):
        valid_k = pvs[k] > 0.0
        wk = 0.0
        mck = 0.0
        for i in range(3):
            sel_k = jnp.where((lv[i]['giota'] == pis[k]) & valid_k, 1.0,
                              0.0)
            wk = wk + jnp.sum(sel_k * lv[i]['oh'], axis=1, keepdims=True)
            mck = mck + jax.lax.dot_general(
                sel_k, (m3, m4, m5)[i][0], (((1,), (1,)), ((), ())),
                preferred_element_type=jnp.float32)
        nv = nv + jnp.sum(wk)
        mcw_rows.append(mck * wk)
    mcw = jnp.concatenate(mcw_rows, axis=0)

    eye12 = jnp.where(
        jax.lax.broadcasted_iota(jnp.int32, (NGT, NGT), 0)
        == jax.lax.broadcasted_iota(jnp.int32, (NGT, NGT), 1), 1.0, 0.0)
    tsel = jnp.concatenate([eye12] * TOPK, axis=0)
    agg = jax.lax.dot_general(tsel, mcw, (((0,), (0,)), ((), ())),
                              preferred_element_type=jnp.float32)
    acc = 0.0
    cross = jnp.zeros((NGT, NM), jnp.float32)
    for c in range(NPIX // PIX_CHUNK):
        pchunk = proto_ref[0, :, PIX_CHUNK * c:PIX_CHUNK * (c + 1)]
        gchunk = gtm_ref[0, :, PIX_CHUNK * c:PIX_CHUNK * (c + 1)]
        pm = jnp.dot(mcw, pchunk, preferred_element_type=jnp.float32)
        sp = jnp.maximum(pm, 0.0) + jnp.log1p(jnp.exp(-jnp.abs(pm)))
        acc = acc + jnp.sum(sp)
        cross = cross + jax.lax.dot_general(
            gchunk, pchunk, (((1,), (1,)), ((), ())),
            preferred_element_type=jnp.float32)
    ln2 = float(np.log1p(np.exp(-0.0)))
    l_msk = (acc - (float(TOPK * NGT) - nv) * ln2 * float(NPIX)
             - jnp.sum(agg * cross)) / float(NPIX)

    oiota = jax.lax.broadcasted_iota(jnp.int32, (1, 128), 1)
    vec = (jnp.where(oiota == 0, l_box, 0.0)
           + jnp.where(oiota == 1, l_cls, 0.0)
           + jnp.where(oiota == 2, l_dfl, 0.0)
           + jnp.where(oiota == 3, l_msk, 0.0)
           + jnp.where(oiota == 4, num_pos, 0.0))
    out_ref[0, :, :] = vec


def kernel(box_p3, box_p4, box_p5, cls_p3, cls_p4, cls_p5, mc_p3, mc_p4,
           mc_p5, proto, gt_boxes, gt_masks):
    B = box_p3.shape[0]
    boxes = [p.reshape(B, 4 * REG_MAX, -1) for p in (box_p3, box_p4, box_p5)]
    clss = [p.reshape(B, 1, -1) for p in (cls_p3, cls_p4, cls_p5)]
    mcs = [p.reshape(B, NM, -1) for p in (mc_p3, mc_p4, mc_p5)]
    proto_r = proto.reshape(B, NM, NPIX)
    gtm_r = gt_masks.reshape(B, NGT, NPIX)
    anchs = [_make_anchor_rows(i) for i in range(3)]

    in_specs = (
        [pl.BlockSpec((8, LEVEL_NA[i]), lambda b: (0, 0)) for i in range(3)]
        + [pl.BlockSpec((1, NGT, 4), lambda b: (b, 0, 0))]
        + [pl.BlockSpec((1, 4 * REG_MAX, LEVEL_NA[i]), lambda b: (b, 0, 0))
           for i in range(3)]
        + [pl.BlockSpec((1, 1, LEVEL_NA[i]), lambda b: (b, 0, 0))
           for i in range(3)]
        + [pl.BlockSpec((1, NM, LEVEL_NA[i]), lambda b: (b, 0, 0))
           for i in range(3)]
        + [pl.BlockSpec((1, NM, NPIX), lambda b: (b, 0, 0)),
           pl.BlockSpec((1, NGT, NPIX), lambda b: (b, 0, 0))]
    )

    out = pl.pallas_call(
        _loss_kernel,
        grid=(B,),
        in_specs=in_specs,
        out_specs=pl.BlockSpec((1, 1, 128), lambda b: (b, 0, 0)),
        out_shape=jax.ShapeDtypeStruct((B, 1, 128), jnp.float32),
        compiler_params=pltpu.CompilerParams(
            dimension_semantics=("parallel",)),
    )(anchs[0], anchs[1], anchs[2], gt_boxes, boxes[0], boxes[1], boxes[2],
      clss[0], clss[1], clss[2], mcs[0], mcs[1], mcs[2], proto_r, gtm_r)

    l_box = jnp.sum(out[:, 0, 0])
    l_cls = jnp.sum(out[:, 0, 1])
    l_dfl = jnp.sum(out[:, 0, 2])
    l_msk = jnp.sum(out[:, 0, 3])
    num_pos = jnp.sum(out[:, 0, 4])
    return (BOX_W * l_box / num_pos + CLS_W * l_cls / num_pos
            + MASK_W * l_msk / num_pos + DFL_W * l_dfl / (num_pos * 4.0))


# speedup vs baseline: 1.0367x; 1.0066x over previous
import numpy as np
import jax
import jax.numpy as jnp
from jax.experimental import pallas as pl
from jax.experimental.pallas import tpu as pltpu

REG_MAX = 16
NM = 32
STRIDES = (8.0, 16.0, 32.0)
LEVELS = ((80, 80), (40, 40), (20, 20))
TOPK = 10
BOX_W, CLS_W, MASK_W, DFL_W = 7.5, 0.5, 2.5, 1.5
EPS = 1e-9
NA = sum(h * w for h, w in LEVELS)
NGT = 12
NPIX = 160 * 160
PIX_CHUNK = 12800
LEVEL_NA = tuple(h * w for h, w in LEVELS)
LEVEL_OFF = (0, LEVEL_NA[0], LEVEL_NA[0] + LEVEL_NA[1])


def _make_anchor_rows(level):
    (h, w), s = LEVELS[level], STRIDES[level]
    rows = np.zeros((8, h * w), dtype=np.float32)
    rows[0] = np.tile((np.arange(w, dtype=np.float32) + 0.5) * s, h)
    rows[1] = np.repeat((np.arange(h, dtype=np.float32) + 0.5) * s, w)
    rows[2] = np.full(h * w, s, dtype=np.float32)
    return jnp.asarray(rows)


def _loss_kernel(a3, a4, a5, gtb_ref, b3, b4, b5, c3, c4, c5, m3, m4, m5,
                 proto_ref, gtm_ref, out_ref):
    g_x1 = gtb_ref[0, :, 0:1]
    g_y1 = gtb_ref[0, :, 1:2]
    g_x2 = gtb_ref[0, :, 2:3]
    g_y2 = gtb_ref[0, :, 3:4]
    area_g = (g_x2 - g_x1) * (g_y2 - g_y1)

    proj = jax.lax.broadcasted_iota(jnp.int32, (REG_MAX, 1), 0).astype(
        jnp.float32)

    lv = []
    for aref, bref, cref in ((a3, b3, c3), (a4, b4, c4), (a5, b5, c5)):
        ax = aref[0:1, :]
        ay = aref[1:2, :]
        stv = aref[2:3, :]
        na = ax.shape[1]

        dists = []
        lses = []
        for k in range(4):
            lg = bref[0, REG_MAX * k:REG_MAX * (k + 1), :]
            mx = jnp.max(lg, axis=0, keepdims=True)
            ex = jnp.exp(lg - mx)
            s = jnp.sum(ex, axis=0, keepdims=True)
            dists.append(jnp.sum((ex / s) * proj, axis=0, keepdims=True))
            lses.append(mx + jnp.log(s))
        x1 = ax - dists[0] * stv
        y1 = ay - dists[1] * stv
        x2 = ax + dists[2] * stv
        y2 = ay + dists[3] * stv

        score = jax.nn.sigmoid(cref[0])

        iw = jnp.clip(jnp.minimum(g_x2, x2) - jnp.maximum(g_x1, x1), 0.0,
                      None)
        ih = jnp.clip(jnp.minimum(g_y2, y2) - jnp.maximum(g_y1, y1), 0.0,
                      None)
        inter = iw * ih
        area_d = (x2 - x1) * (y2 - y1)
        ious = inter / (area_g + area_d - inter + EPS)

        i2 = ious * ious
        align = score * (i2 * i2 * i2)
        in_gt = ((ax > g_x1) & (ax < g_x2) & (ay > g_y1) & (ay < g_y2))
        metric = jnp.where(in_gt, align, 0.0)
        lane_iota = jax.lax.broadcasted_iota(jnp.int32, (NGT, na), 1)
        lv.append(dict(ax=ax, ay=ay, stv=stv, lses=lses, x1=x1, y1=y1,
                       x2=x2, y2=y2, area_d=area_d, ious=ious,
                       metric=metric, giota=lane_iota + LEVEL_OFF[len(lv)],
                       na=na))

    pvs = []
    pis = []
    pv = jnp.full((NGT, 1), jnp.inf, jnp.float32)
    pi = jnp.full((NGT, 1), -1, jnp.int32)
    for _ in range(TOPK):
        ems = []
        for i in range(3):
            m = lv[i]['metric']
            elig = (m < pv) | ((m == pv) & (lv[i]['giota'] > pi))
            ems.append(jnp.where(elig, m, -1.0))
        mx = jnp.maximum(
            jnp.maximum(jnp.max(ems[0], axis=1, keepdims=True),
                        jnp.max(ems[1], axis=1, keepdims=True)),
            jnp.max(ems[2], axis=1, keepdims=True))
        pos = jnp.minimum(
            jnp.minimum(
                jnp.min(jnp.where(ems[0] == mx, lv[0]['giota'], NA),
                        axis=1, keepdims=True),
                jnp.min(jnp.where(ems[1] == mx, lv[1]['giota'], NA),
                        axis=1, keepdims=True)),
            jnp.min(jnp.where(ems[2] == mx, lv[2]['giota'], NA),
                    axis=1, keepdims=True))
        pv, pi = mx, pos
        pvs.append(pv)
        pis.append(pi)
    pv10, pi10 = pvs[-1], pis[-1]

    l_box = 0.0
    l_cls = 0.0
    l_dfl = 0.0
    num_pos = 0.0
    for i in range(3):
        l = lv[i]
        na = l['na']
        m = l['metric']
        selected = (m > 0.0) & ((m > pv10)
                                | ((m == pv10) & (l['giota'] <= pi10)))
        fgf = jnp.max(jnp.where(selected, 1.0, 0.0), axis=0,
                      keepdims=True)
        num_pos = num_pos + jnp.sum(fgf)

        iou_m = jnp.where(selected, l['ious'], -1.0)
        best = iou_m[0:1, :]
        bidx = jnp.zeros((1, na), jnp.int32)
        for g in range(1, NGT):
            v = iou_m[g:g + 1, :]
            take = v > best
            best = jnp.where(take, v, best)
            bidx = jnp.where(take, g, bidx)
        sub_iota = jax.lax.broadcasted_iota(jnp.int32, (NGT, na), 0)
        oh = jnp.where(sub_iota == bidx, 1.0, 0.0)
        l['oh'] = oh

        bg_x1 = jnp.sum(oh * g_x1, axis=0, keepdims=True)
        bg_y1 = jnp.sum(oh * g_y1, axis=0, keepdims=True)
        bg_x2 = jnp.sum(oh * g_x2, axis=0, keepdims=True)
        bg_y2 = jnp.sum(oh * g_y2, axis=0, keepdims=True)

        eiw = jnp.clip(jnp.minimum(bg_x2, l['x2'])
                       - jnp.maximum(bg_x1, l['x1']), 0.0, None)
        eih = jnp.clip(jnp.minimum(bg_y2, l['y2'])
                       - jnp.maximum(bg_y1, l['y1']), 0.0, None)
        einter = eiw * eih
        area_b = (bg_x2 - bg_x1) * (bg_y2 - bg_y1)
        iou_e = einter / (l['area_d'] + area_b - einter + EPS)
        l_box = l_box + jnp.sum((1.0 - iou_e) * fgf)

        x = (c3, c4, c5)[i][0]
        tgt = jnp.clip(iou_e, 0.0, 1.0)
        bce = (jnp.maximum(x, 0.0) - x * tgt
               + jnp.log1p(jnp.exp(-jnp.abs(x))))
        l_cls = l_cls + jnp.sum(bce * fgf)

        bref = (b3, b4, b5)[i]
        tvals = ((l['ax'] - bg_x1) / l['stv'], (l['ay'] - bg_y1) / l['stv'],
                 (bg_x2 - l['ax']) / l['stv'], (bg_y2 - l['ay']) / l['stv'])
        bin_iota = jax.lax.broadcasted_iota(jnp.int32, (REG_MAX, na), 0)
        for k in range(4):
            t = jnp.clip(tvals[k], 0.0, REG_MAX - 1e-6)
            tl = t.astype(jnp.int32)
            tr = jnp.minimum(tl + 1, REG_MAX - 1)
            at_b = tl >= REG_MAX - 1
            tr = jnp.where(at_b, tl, tr)
            wr = t - tl.astype(jnp.float32)
            wl = 1.0 - wr
            wr = jnp.where(at_b, 0.0, wr)
            wl = jnp.where(at_b, 1.0, wl)
            lgk = bref[0, REG_MAX * k:REG_MAX * (k + 1), :]
            pick_l = jnp.sum(jnp.where(bin_iota == tl, lgk, 0.0), axis=0,
                             keepdims=True)
            pick_r = jnp.sum(jnp.where(bin_iota == tr, lgk, 0.0), axis=0,
                             keepdims=True)
            lse = l['lses'][k]
            l_dfl = l_dfl + jnp.sum(((lse - pick_l) * wl
                                     + (lse - pick_r) * wr) * fgf)

    mcw_rows = []
    nv = 0.0
    for k in range(TOPK):
        valid_k = pvs[k] > 0.0
        wk = 0.0
        mck = 0.0
        for i in range(3):
            sel_k = jnp.where((lv[i]['giota'] == pis[k]) & valid_k, 1.0,
                              0.0)
            wk = wk + jnp.sum(sel_k * lv[i]['oh'], axis=1, keepdims=True)
            mck = mck + jax.lax.dot_general(
                sel_k, (m3, m4, m5)[i][0], (((1,), (1,)), ((), ())),
                preferred_element_type=jnp.float32)
        nv = nv + jnp.sum(wk)
        mcw_rows.append(mck * wk)
    mcw = jnp.concatenate(mcw_rows, axis=0)

    eye12 = jnp.where(
        jax.lax.broadcasted_iota(jnp.int32, (NGT, NGT), 0)
        == jax.lax.broadcasted_iota(jnp.int32, (NGT, NGT), 1), 1.0, 0.0)
    tsel = jnp.concatenate([eye12] * TOPK, axis=0)
    agg = jax.lax.dot_general(tsel, mcw, (((0,), (0,)), ((), ())),
                              preferred_element_type=jnp.float32)
    acc = 0.0
    cross = jnp.zeros((NGT, NM), jnp.float32)
    for c in range(NPIX // PIX_CHUNK):
        pchunk = proto_ref[0, :, PIX_CHUNK * c:PIX_CHUNK * (c + 1)]
        gchunk = gtm_ref[0, :, PIX_CHUNK * c:PIX_CHUNK * (c + 1)]
        pm = jnp.dot(mcw, pchunk, preferred_element_type=jnp.float32)
        sp = jnp.maximum(pm, 0.0) + jnp.log1p(jnp.exp(-jnp.abs(pm)))
        acc = acc + jnp.sum(sp)
        cross = cross + jax.lax.dot_general(
            gchunk, pchunk, (((1,), (1,)), ((), ())),
            preferred_element_type=jnp.float32)
    ln2 = float(np.log1p(np.exp(-0.0)))
    l_msk = (acc - (float(TOPK * NGT) - nv) * ln2 * float(NPIX)
             - jnp.sum(agg * cross)) / float(NPIX)

    oiota = jax.lax.broadcasted_iota(jnp.int32, (1, 128), 1)
    vec = (jnp.where(oiota == 0, l_box, 0.0)
           + jnp.where(oiota == 1, l_cls, 0.0)
           + jnp.where(oiota == 2, l_dfl, 0.0)
           + jnp.where(oiota == 3, l_msk, 0.0)
           + jnp.where(oiota == 4, num_pos, 0.0))
    out_ref[0, :, :] = vec


def kernel(box_p3, box_p4, box_p5, cls_p3, cls_p4, cls_p5, mc_p3, mc_p4,
           mc_p5, proto, gt_boxes, gt_masks):
    B = box_p3.shape[0]
    boxes = [p.reshape(B, 4 * REG_MAX, -1) for p in (box_p3, box_p4, box_p5)]
    clss = [p.reshape(B, 1, -1) for p in (cls_p3, cls_p4, cls_p5)]
    mcs = [p.reshape(B, NM, -1) for p in (mc_p3, mc_p4, mc_p5)]
    proto_r = proto.reshape(B, NM, NPIX)
    gtm_r = gt_masks.reshape(B, NGT, NPIX)
    anchs = [_make_anchor_rows(i) for i in range(3)]

    in_specs = (
        [pl.BlockSpec((8, LEVEL_NA[i]), lambda b: (0, 0)) for i in range(3)]
        + [pl.BlockSpec((1, NGT, 4), lambda b: (b, 0, 0))]
        + [pl.BlockSpec((1, 4 * REG_MAX, LEVEL_NA[i]), lambda b: (b, 0, 0))
           for i in range(3)]
        + [pl.BlockSpec((1, 1, LEVEL_NA[i]), lambda b: (b, 0, 0))
           for i in range(3)]
        + [pl.BlockSpec((1, NM, LEVEL_NA[i]), lambda b: (b, 0, 0))
           for i in range(3)]
        + [pl.BlockSpec((1, NM, NPIX), lambda b: (b, 0, 0)),
           pl.BlockSpec((1, NGT, NPIX), lambda b: (b, 0, 0))]
    )

    out = pl.pallas_call(
        _loss_kernel,
        grid=(B,),
        in_specs=in_specs,
        out_specs=pl.BlockSpec((1, 1, 128), lambda b: (b, 0, 0)),
        out_shape=jax.ShapeDtypeStruct((B, 1, 128), jnp.float32),
        compiler_params=pltpu.CompilerParams(
            dimension_semantics=("parallel",)),
    )(anchs[0], anchs[1], anchs[2], gt_boxes, boxes[0], boxes[1], boxes[2],
      clss[0], clss[1], clss[2], mcs[0], mcs[1], mcs[2], proto_r, gtm_r)

    l_box = jnp.sum(out[:, 0, 0])
    l_cls = jnp.sum(out[:, 0, 1])
    l_dfl = jnp.sum(out[:, 0, 2])
    l_msk = jnp.sum(out[:, 0, 3])
    num_pos = jnp.sum(out[:, 0, 4])
    return (BOX_W * l_box / num_pos + CLS_W * l_cls / num_pos
            + MASK_W * l_msk / num_pos + DFL_W * l_dfl / (num_pos * 4.0))
